# baseline (speedup 1.0000x reference)
import functools

import jax
import jax.numpy as jnp
from jax import lax
from jax.experimental import pallas as pl
from jax.experimental.pallas import tpu as pltpu
from jax.experimental.pallas import tpu_sc as plsc

_B = 16384
_NF = 5
_ED = 8
_Q = _NF * _ED
_NW = 32
_BPW = _B // _NW
_HPW = _BPW // 128
_RPW = _HPW * _Q

_TC_BS = 16384
_TC_RB = _TC_BS // 128 * _Q


def _sc_gather(idxs, tabT, offs):
    mesh = plsc.VectorSubcoreMesh(core_axis_name="c", subcore_axis_name="s")

    @functools.partial(
        pl.kernel,
        mesh=mesh,
        out_type=jax.ShapeDtypeStruct((_B // 128 * _Q * 128,), jnp.float32),
        scratch_types=[
            pltpu.VMEM((_NF * _BPW,), jnp.int32),
            pltpu.VMEM((_ED, 40), jnp.float32),
            pltpu.VMEM((_RPW * 128,), jnp.float32),
            pltpu.SemaphoreType.DMA,
        ],
        compiler_params=pltpu.CompilerParams(
            use_tc_tiling_on_sc=False, needs_layout_passes=False),
    )
    def k(i0, i1, i2, i3, i4, tT, out_hbm, idx_v, tab_v, buf_v, sem):
        wid = lax.axis_index("s") * 2 + lax.axis_index("c")
        base = wid * _BPW
        stage = [
            pltpu.async_copy(ih.at[pl.ds(base, _BPW)],
                             idx_v.at[pl.ds(f * _BPW, _BPW)], sem)
            for f, ih in enumerate((i0, i1, i2, i3, i4))
        ] + [pltpu.async_copy(tT, tab_v, sem)]
        for cp in stage:
            cp.wait()

        cvecs = [jnp.full((16,), c, jnp.int32) for c in range(_ED)]

        for f in range(_NF):
            for hi in range(_HPW):

                def body(p, _, f=f, hi=hi):
                    for u in range(2):
                        rows = idx_v[pl.ds(
                            f * _BPW + hi * 128 + p * 32 + u * 16, 16)]
                        rows = rows + offs[f]
                        for c in range(_ED):
                            vals = plsc.load_gather(tab_v, [cvecs[c], rows])
                            buf_v[pl.ds(
                                (hi * _Q + f * _ED + c) * 128
                                + p * 32 + u * 16, 16
                            )] = vals
                    return _

                lax.fori_loop(0, 4, body, None)

        pltpu.async_copy(
            buf_v, out_hbm.at[pl.ds(wid * _RPW * 128, _RPW * 128)],
            sem).wait()

    return k(*idxs, tabT)


def _tc_mlp(xq, W1, b1, ln_g, ln_b, W2, b2):
    nb = _B // _TC_BS
    dn = (((0,), (0,)), ((), ()))

    def body(x_ref, w1_ref, b1_ref, g_ref, be_ref, w2_ref, b2_ref, o_ref):
        xq2 = x_ref[...].reshape(_TC_RB, 128)
        hs = [
            lax.dot_general(xq2[k * _Q:(k + 1) * _Q, :], w1_ref[...], dn,
                            preferred_element_type=jnp.float32)
            for k in range(_TC_BS // 128)
        ]
        h = jnp.concatenate(hs, axis=0)
        h = h + b1_ref[...]
        mu = jnp.mean(h, axis=1, keepdims=True)
        var = jnp.mean((h - mu) ** 2, axis=1, keepdims=True)
        h = (h - mu) * lax.rsqrt(var + 1e-5) * g_ref[...] + be_ref[...]
        h = jnp.maximum(h, 0.0)
        o_ref[...] = (
            jnp.dot(h, w2_ref[...], preferred_element_type=jnp.float32)
            + b2_ref[...]
        )

    return pl.pallas_call(
        body,
        grid=(nb,),
        in_specs=[
            pl.BlockSpec((_TC_RB * 128,), lambda i: (i,)),
            pl.BlockSpec((_Q, 128), lambda i: (0, 0)),
            pl.BlockSpec((128,), lambda i: (0,)),
            pl.BlockSpec((128,), lambda i: (0,)),
            pl.BlockSpec((128,), lambda i: (0,)),
            pl.BlockSpec((128, 64), lambda i: (0, 0)),
            pl.BlockSpec((64,), lambda i: (0,)),
        ],
        out_specs=pl.BlockSpec((_TC_BS, 64), lambda i: (i, 0)),
        out_shape=jax.ShapeDtypeStruct((_B, 64), jnp.float32),
    )(xq, W1, b1, ln_g, ln_b, W2, b2)


def kernel(admission_type, admission_location, drg_type, drg_severity,
           drg_mortality, emb_admission_type, emb_admission_location,
           emb_drg_type, emb_drg_severity, emb_drg_mortality,
           W1, b1, ln_g, ln_b, W2, b2):
    idxs = [admission_type.astype(jnp.int32),
            admission_location.astype(jnp.int32),
            drg_type.astype(jnp.int32),
            drg_severity.astype(jnp.int32),
            drg_mortality.astype(jnp.int32)]
    tabs = [emb_admission_type, emb_admission_location, emb_drg_type,
            emb_drg_severity, emb_drg_mortality]
    sizes = [t.shape[0] for t in tabs]
    offs = [sum(sizes[:f]) for f in range(_NF)]
    tabT = jnp.concatenate(tabs, axis=0)
    tabT = jnp.pad(tabT, ((0, 40 - tabT.shape[0]), (0, 0))).T

    xq = _sc_gather(idxs, tabT, offs)
    return _tc_mlp(xq, W1, b1, ln_g, ln_b, W2, b2)


# speedup vs baseline: 1.0684x; 1.0684x over previous
import functools

import jax
import jax.numpy as jnp
from jax import lax
from jax.experimental import pallas as pl
from jax.experimental.pallas import tpu as pltpu
from jax.experimental.pallas import tpu_sc as plsc

_B = 16384
_NF = 5
_ED = 8
_Q = _NF * _ED
_NW = 32
_BPW = _B // _NW
_HPW = _BPW // 128
_RPW = _HPW * _Q

_TC_BS = 16384
_TC_RB = _TC_BS // 128 * _Q


def _sc_gather(idxs, tabT, offs):
    mesh = plsc.VectorSubcoreMesh(core_axis_name="c", subcore_axis_name="s")

    @functools.partial(
        pl.kernel,
        mesh=mesh,
        out_type=jax.ShapeDtypeStruct((_B // 128 * _Q * 128,), jnp.float32),
        scratch_types=[
            pltpu.VMEM((_NF * _BPW,), jnp.int32),
            pltpu.VMEM((_ED, 40), jnp.float32),
            pltpu.VMEM((_RPW * 128,), jnp.float32),
            pltpu.SemaphoreType.DMA,
        ],
        compiler_params=pltpu.CompilerParams(
            use_tc_tiling_on_sc=False, needs_layout_passes=False),
    )
    def k(i0, i1, i2, i3, i4, tT, out_hbm, idx_v, tab_v, buf_v, sem):
        wid = lax.axis_index("s") * 2 + lax.axis_index("c")
        base = wid * _BPW
        stage = [
            pltpu.async_copy(ih.at[pl.ds(base, _BPW)],
                             idx_v.at[pl.ds(f * _BPW, _BPW)], sem)
            for f, ih in enumerate((i0, i1, i2, i3, i4))
        ] + [pltpu.async_copy(tT, tab_v, sem)]
        for cp in stage:
            cp.wait()

        cvecs = [jnp.full((16,), c, jnp.int32) for c in range(_ED)]

        for f in range(_NF):
            for hi in range(_HPW):

                def body(p, _, f=f, hi=hi):
                    rows = idx_v[pl.ds(f * _BPW + hi * 128 + p * 16, 16)]
                    rows = rows + offs[f]
                    for c in range(_ED):
                        vals = plsc.load_gather(tab_v, [cvecs[c], rows])
                        buf_v[pl.ds(
                            (hi * _Q + f * _ED + c) * 128 + p * 16, 16
                        )] = vals
                    return _

                lax.fori_loop(0, 8, body, None)

        pltpu.async_copy(
            buf_v, out_hbm.at[pl.ds(wid * _RPW * 128, _RPW * 128)],
            sem).wait()

    return k(*idxs, tabT)


def _tc_mlp(xq, W1, b1, ln_g, ln_b, W2, b2):
    nb = _B // _TC_BS
    dn = (((0,), (0,)), ((), ()))

    def body(x_ref, w1_ref, b1_ref, g_ref, be_ref, w2_ref, b2_ref, o_ref):
        xq2 = x_ref[...].reshape(_TC_RB, 128)
        hs = [
            lax.dot_general(xq2[k * _Q:(k + 1) * _Q, :], w1_ref[...], dn,
                            preferred_element_type=jnp.float32)
            for k in range(_TC_BS // 128)
        ]
        h = jnp.concatenate(hs, axis=0)
        h = h + b1_ref[...]
        mu = jnp.mean(h, axis=1, keepdims=True)
        var = jnp.mean((h - mu) ** 2, axis=1, keepdims=True)
        h = (h - mu) * lax.rsqrt(var + 1e-5) * g_ref[...] + be_ref[...]
        h = jnp.maximum(h, 0.0)
        o_ref[...] = (
            jnp.dot(h, w2_ref[...], preferred_element_type=jnp.float32)
            + b2_ref[...]
        )

    return pl.pallas_call(
        body,
        grid=(nb,),
        in_specs=[
            pl.BlockSpec((_TC_RB * 128,), lambda i: (i,)),
            pl.BlockSpec((_Q, 128), lambda i: (0, 0)),
            pl.BlockSpec((128,), lambda i: (0,)),
            pl.BlockSpec((128,), lambda i: (0,)),
            pl.BlockSpec((128,), lambda i: (0,)),
            pl.BlockSpec((128, 64), lambda i: (0, 0)),
            pl.BlockSpec((64,), lambda i: (0,)),
        ],
        out_specs=pl.BlockSpec((_TC_BS, 64), lambda i: (i, 0)),
        out_shape=jax.ShapeDtypeStruct((_B, 64), jnp.float32),
    )(xq, W1, b1, ln_g, ln_b, W2, b2)


def kernel(admission_type, admission_location, drg_type, drg_severity,
           drg_mortality, emb_admission_type, emb_admission_location,
           emb_drg_type, emb_drg_severity, emb_drg_mortality,
           W1, b1, ln_g, ln_b, W2, b2):
    idxs = [admission_type.astype(jnp.int32),
            admission_location.astype(jnp.int32),
            drg_type.astype(jnp.int32),
            drg_severity.astype(jnp.int32),
            drg_mortality.astype(jnp.int32)]
    tabs = [emb_admission_type, emb_admission_location, emb_drg_type,
            emb_drg_severity, emb_drg_mortality]
    sizes = [t.shape[0] for t in tabs]
    offs = [sum(sizes[:f]) for f in range(_NF)]
    tabT = jnp.concatenate(tabs, axis=0)
    tabT = jnp.pad(tabT, ((0, 40 - tabT.shape[0]), (0, 0))).T

    xq = _sc_gather(idxs, tabT, offs)
    return _tc_mlp(xq, W1, b1, ln_g, ln_b, W2, b2)
